# baseline (speedup 1.0000x reference)
import functools

import jax
import jax.numpy as jnp
from jax import lax
from jax.experimental import pallas as pl
from jax.experimental.pallas import tpu as pltpu
from jax.experimental.pallas import tpu_sc as plsc

_L = 16
_CH = 128


def kernel(state, action, rewardMatrix):
    S, A = rewardMatrix.shape
    B = state.shape[0]
    rm_flat = rewardMatrix.reshape(S * A)

    info = plsc.get_sparse_core_info()
    nw = info.num_cores * info.num_subcores
    b_per_w = B // nw
    n_ch = b_per_w // _CH
    nc = info.num_cores

    mesh = plsc.VectorSubcoreMesh(core_axis_name="c", subcore_axis_name="s")

    @functools.partial(
        pl.kernel,
        out_type=jax.ShapeDtypeStruct((B,), jnp.float32),
        mesh=mesh,
        scratch_types=[
            pltpu.VMEM((b_per_w,), jnp.int32),
            pltpu.VMEM((b_per_w,), jnp.int32),
            pltpu.VMEM((b_per_w,), jnp.int32),
            pltpu.VMEM((b_per_w,), jnp.float32),
            pltpu.SemaphoreType.DMA,
        ],
    )
    def body(state_hbm, action_hbm, rm_hbm, out_hbm, st_v, ac_v, idx_v, val_v, sem):
        wid = lax.axis_index("s") * nc + lax.axis_index("c")
        base = wid * b_per_w
        pltpu.sync_copy(state_hbm.at[pl.ds(base, b_per_w)], st_v)
        pltpu.sync_copy(action_hbm.at[pl.ds(base, b_per_w)], ac_v)
        for k in range(b_per_w // _L):
            sl = pl.ds(k * _L, _L)
            idx_v[sl] = st_v[sl] * A + ac_v[sl]
        copies = [
            pltpu.async_copy(
                rm_hbm.at[idx_v.at[pl.ds(j * _CH, _CH)]],
                val_v.at[pl.ds(j * _CH, _CH)],
                sem,
            )
            for j in range(n_ch)
        ]
        for cp in copies:
            cp.wait()
        pltpu.sync_copy(val_v, out_hbm.at[pl.ds(base, b_per_w)])

    return body(state, action, rm_flat)
